# baseline (speedup 1.0000x reference)
import functools

import jax
import jax.numpy as jnp
from jax import lax
from jax.experimental import pallas as pl
from jax.experimental.pallas import tpu as pltpu
from jax.experimental.pallas import tpu_sc as plsc

_VOCAB = 100000
_HIDDEN = 768
_LANES = 16
_NSLICE = _HIDDEN // _LANES
_EPS = 1e-12

_NTOK = 8192
_NW = 32
_TPW = _NTOK // _NW
_CHUNK = 128
_NCHUNK = _TPW // _CHUNK


def _rsqrt_vec(v):
    i = plsc.bitcast(v, jnp.int32)
    y = plsc.bitcast(jnp.int32(0x5F3759DF) - (i >> 1), jnp.float32)
    for _ in range(3):
        y = y * (1.5 - 0.5 * v * y * y)
    return y


def _make_kernel():
    mesh = plsc.VectorSubcoreMesh(core_axis_name="c", subcore_axis_name="s")

    @functools.partial(
        pl.kernel,
        mesh=mesh,
        compiler_params=pltpu.CompilerParams(needs_layout_passes=False),
        out_type=jax.ShapeDtypeStruct((_NTOK, _HIDDEN), jnp.float32),
        scratch_types=[
            pltpu.VMEM((_CHUNK,), jnp.int32),
            pltpu.VMEM((_TPW,), jnp.int32),
            pltpu.VMEM((_CHUNK, _HIDDEN), jnp.float32),
            pltpu.VMEM((2, _HIDDEN), jnp.float32),
            pltpu.VMEM((_HIDDEN,), jnp.float32),
            pltpu.VMEM((_HIDDEN,), jnp.float32),
            pltpu.SemaphoreType.DMA,
        ],
    )
    def emb_kernel(ids_hbm, ttids_hbm, wtab_hbm, tttab_hbm, gamma_hbm,
                   beta_hbm, out_hbm, idx_v, ttid_v, rows_v, tt_v, g_v,
                   b_v, sem):
        nc = 2
        wid = lax.axis_index("s") * nc + lax.axis_index("c")
        tbase = wid * _TPW

        pltpu.sync_copy(tttab_hbm, tt_v)
        pltpu.sync_copy(gamma_hbm, g_v)
        pltpu.sync_copy(beta_hbm, b_v)
        pltpu.sync_copy(ttids_hbm.at[pl.ds(tbase, _TPW)], ttid_v)

        def make_row_body(c):
          def row_body(r, carry):
            lane_idx = jnp.full((_LANES,), c * _CHUNK + r, jnp.int32)
            tfv = plsc.load_gather(ttid_v, [lane_idx])
            tf = tfv.astype(jnp.float32)

            acc = jnp.zeros((_LANES,), jnp.float32)
            accq = jnp.zeros((_LANES,), jnp.float32)
            for s in range(_NSLICE):
                sl = pl.ds(s * _LANES, _LANES)
                w = rows_v[r, sl]
                tt = tt_v[0, sl] + tf * (tt_v[1, sl] - tt_v[0, sl])
                x = w + tt
                rows_v[r, sl] = x
                acc = acc + x
                accq = accq + x * x
            ssum = jnp.sum(acc)
            sqsum = jnp.sum(accq)
            mean = ssum * (1.0 / _HIDDEN)
            var = jnp.maximum(sqsum * (1.0 / _HIDDEN) - mean * mean, 0.0)
            meanv = jnp.full((_LANES,), mean)
            rv = _rsqrt_vec(jnp.full((_LANES,), var + _EPS))
            for s in range(_NSLICE):
                sl = pl.ds(s * _LANES, _LANES)
                x = rows_v[r, sl]
                rows_v[r, sl] = (x - meanv) * rv * g_v[sl] + b_v[sl]
            return carry
          return row_body

        for c in range(_NCHUNK):
            cbase = tbase + c * _CHUNK
            pltpu.sync_copy(ids_hbm.at[pl.ds(cbase, _CHUNK)], idx_v)
            pltpu.async_copy(wtab_hbm.at[idx_v], rows_v, sem).wait()
            lax.fori_loop(0, _CHUNK, make_row_body(c), jnp.int32(0))
            pltpu.sync_copy(rows_v, out_hbm.at[pl.ds(cbase, _CHUNK)])

    return emb_kernel


_EMB_KERNEL = _make_kernel()


def kernel(input_ids, token_type_ids, word_emb, token_type_emb, ln_gamma,
           ln_beta):
    b, l = input_ids.shape
    ids = input_ids.reshape(b * l).astype(jnp.int32)
    ttids = token_type_ids.reshape(b * l).astype(jnp.int32)
    out = _EMB_KERNEL(ids, ttids, word_emb, token_type_emb, ln_gamma,
                      ln_beta)
    return out.reshape(b, l, word_emb.shape[1])


# speedup vs baseline: 2.7511x; 2.7511x over previous
import functools

import jax
import jax.numpy as jnp
from jax import lax
from jax.experimental import pallas as pl
from jax.experimental.pallas import tpu as pltpu
from jax.experimental.pallas import tpu_sc as plsc

_HIDDEN = 768
_LANES = 16
_NSLICE = _HIDDEN // _LANES
_EPS = 1e-12

_NTOK = 8192
_NW = 32
_TPW = _NTOK // _NW
_CHUNK = 64
_NCHUNK = _TPW // _CHUNK
_G = 8
_NG = _CHUNK // _G


def _rsqrt_vec(v):
    i = plsc.bitcast(v, jnp.int32)
    y = plsc.bitcast(jnp.int32(0x5F3759DF) - (i >> 1), jnp.float32)
    for _ in range(3):
        y = y * (1.5 - 0.5 * v * y * y)
    return y


def _make_kernel():
    mesh = plsc.VectorSubcoreMesh(core_axis_name="c", subcore_axis_name="s")

    @functools.partial(
        pl.kernel,
        mesh=mesh,
        compiler_params=pltpu.CompilerParams(needs_layout_passes=False),
        out_type=jax.ShapeDtypeStruct((_NTOK, _HIDDEN), jnp.float32),
        scratch_types=[
            pltpu.VMEM((_TPW,), jnp.int32),
            pltpu.VMEM((_TPW,), jnp.int32),
            pltpu.VMEM((2, _CHUNK, _HIDDEN), jnp.float32),
            pltpu.VMEM((2, _HIDDEN), jnp.float32),
            pltpu.VMEM((_HIDDEN,), jnp.float32),
            pltpu.VMEM((_HIDDEN,), jnp.float32),
            pltpu.SemaphoreType.DMA,
            pltpu.SemaphoreType.DMA,
            pltpu.SemaphoreType.DMA,
            pltpu.SemaphoreType.DMA,
        ],
    )
    def emb_kernel(ids_hbm, ttids_hbm, wtab_hbm, tttab_hbm, gamma_hbm,
                   beta_hbm, out_hbm, idx_v, ttid_v, rows_v, tt_v, g_v,
                   b_v, gsem0, gsem1, ssem0, ssem1):
        nc = 2
        wid = lax.axis_index("s") * nc + lax.axis_index("c")
        tbase = wid * _TPW
        gsem = (gsem0, gsem1)
        ssem = (ssem0, ssem1)

        pltpu.sync_copy(tttab_hbm, tt_v)
        pltpu.sync_copy(gamma_hbm, g_v)
        pltpu.sync_copy(beta_hbm, b_v)
        pltpu.sync_copy(ttids_hbm.at[pl.ds(tbase, _TPW)], ttid_v)
        pltpu.sync_copy(ids_hbm.at[pl.ds(tbase, _TPW)], idx_v)

        def diff_body(s, carry):
            sl = pl.ds(s * _LANES, _LANES)
            tt_v[1, sl] = tt_v[1, sl] - tt_v[0, sl]
            return carry

        lax.fori_loop(0, _NSLICE, diff_body, jnp.int32(0))

        def start_gather(c, b):
            return pltpu.async_copy(
                wtab_hbm.at[idx_v.at[pl.ds(c * _CHUNK, _CHUNK)]],
                rows_v.at[b], gsem[b])

        def compute(c, b):
            def group_body(g, carry):
                rbase = g * _G
                tfs = []
                for r in range(_G):
                    lane_idx = jnp.full((_LANES,), c * _CHUNK + rbase + r,
                                        jnp.int32)
                    tfs.append(
                        plsc.load_gather(ttid_v,
                                         [lane_idx]).astype(jnp.float32))

                zeros = jnp.zeros((_LANES,), jnp.float32)

                def s1_body(s, acc):
                    accs, accqs = acc
                    sl = pl.ds(s * _LANES, _LANES)
                    tt0 = tt_v[0, sl]
                    ttd = tt_v[1, sl]
                    new_a = []
                    new_q = []
                    for r in range(_G):
                        x = rows_v[b, rbase + r, sl] + tt0 + tfs[r] * ttd
                        rows_v[b, rbase + r, sl] = x
                        new_a.append(accs[r] + x)
                        new_q.append(accqs[r] + x * x)
                    return tuple(new_a), tuple(new_q)

                accs, accqs = lax.fori_loop(
                    0, _NSLICE, s1_body,
                    (tuple([zeros] * _G), tuple([zeros] * _G)))

                means = []
                rvs = []
                for r in range(_G):
                    mean = jnp.sum(accs[r]) * (1.0 / _HIDDEN)
                    var = jnp.maximum(
                        jnp.sum(accqs[r]) * (1.0 / _HIDDEN) - mean * mean,
                        0.0)
                    means.append(jnp.full((_LANES,), mean))
                    rvs.append(_rsqrt_vec(jnp.full((_LANES,), var + _EPS)))

                def s2_body(s, carry2):
                    sl = pl.ds(s * _LANES, _LANES)
                    gm = g_v[sl]
                    bt = b_v[sl]
                    for r in range(_G):
                        x = rows_v[b, rbase + r, sl]
                        rows_v[b, rbase + r, sl] = (
                            (x - means[r]) * rvs[r] * gm + bt)
                    return carry2

                lax.fori_loop(0, _NSLICE, s2_body, jnp.int32(0))
                return carry

            lax.fori_loop(0, _NG, group_body, jnp.int32(0))

        gh = [None] * _NCHUNK
        sh = [None] * _NCHUNK
        gh[0] = start_gather(0, 0)
        for c in range(_NCHUNK):
            b = c & 1
            nb = b ^ 1
            if c + 1 < _NCHUNK:
                if c - 1 >= 0:
                    sh[c - 1].wait()
                gh[c + 1] = start_gather(c + 1, nb)
            gh[c].wait()
            compute(c, b)
            sh[c] = pltpu.async_copy(
                rows_v.at[b],
                out_hbm.at[pl.ds(tbase + c * _CHUNK, _CHUNK)], ssem[b])
        sh[_NCHUNK - 2].wait()
        sh[_NCHUNK - 1].wait()

    return emb_kernel


_EMB_KERNEL = _make_kernel()


def kernel(input_ids, token_type_ids, word_emb, token_type_emb, ln_gamma,
           ln_beta):
    b, l = input_ids.shape
    ids = input_ids.reshape(b * l).astype(jnp.int32)
    ttids = token_type_ids.reshape(b * l).astype(jnp.int32)
    out = _EMB_KERNEL(ids, ttids, word_emb, token_type_emb, ln_gamma,
                      ln_beta)
    return out.reshape(b, l, word_emb.shape[1])


# speedup vs baseline: 3.7630x; 1.3678x over previous
import functools

import jax
import jax.numpy as jnp
from jax import lax
from jax.experimental import pallas as pl
from jax.experimental.pallas import tpu as pltpu
from jax.experimental.pallas import tpu_sc as plsc

_HIDDEN = 768
_LANES = 16
_NSLICE = _HIDDEN // _LANES
_EPS = 1e-12

_NTOK = 8192
_NW = 32
_TPW = _NTOK // _NW
_CHUNK = 64
_NCHUNK = _TPW // _CHUNK
_G = 8
_NG = _CHUNK // _G


def _rsqrt_vec(v):
    i = plsc.bitcast(v, jnp.int32)
    y = plsc.bitcast(jnp.int32(0x5F3759DF) - (i >> 1), jnp.float32)
    for _ in range(3):
        y = y * (1.5 - 0.5 * v * y * y)
    return y


def _make_kernel():
    mesh = plsc.VectorSubcoreMesh(core_axis_name="c", subcore_axis_name="s")

    @functools.partial(
        pl.kernel,
        mesh=mesh,
        compiler_params=pltpu.CompilerParams(needs_layout_passes=False),
        out_type=jax.ShapeDtypeStruct((_NTOK, _HIDDEN), jnp.float32),
        scratch_types=[
            pltpu.VMEM((_TPW,), jnp.int32),
            pltpu.VMEM((_TPW,), jnp.int32),
            pltpu.VMEM((2, _CHUNK, _HIDDEN), jnp.float32),
            pltpu.VMEM((2, _HIDDEN), jnp.float32),
            pltpu.VMEM((_HIDDEN,), jnp.float32),
            pltpu.VMEM((_HIDDEN,), jnp.float32),
            pltpu.SemaphoreType.DMA,
            pltpu.SemaphoreType.DMA,
            pltpu.SemaphoreType.DMA,
            pltpu.SemaphoreType.DMA,
        ],
    )
    def emb_kernel(ids_hbm, ttids_hbm, wtab_hbm, tttab_hbm, gamma_hbm,
                   beta_hbm, out_hbm, idx_v, ttid_v, rows_v, tt_v, g_v,
                   b_v, gsem0, gsem1, ssem0, ssem1):
        nc = 2
        wid = lax.axis_index("s") * nc + lax.axis_index("c")
        tbase = wid * _TPW
        gsem = (gsem0, gsem1)
        ssem = (ssem0, ssem1)

        pltpu.sync_copy(tttab_hbm, tt_v)
        pltpu.sync_copy(gamma_hbm, g_v)
        pltpu.sync_copy(beta_hbm, b_v)
        pltpu.sync_copy(ttids_hbm.at[pl.ds(tbase, _TPW)], ttid_v)
        pltpu.sync_copy(ids_hbm.at[pl.ds(tbase, _TPW)], idx_v)

        def start_gather(c, b):
            return pltpu.async_copy(
                wtab_hbm.at[idx_v.at[pl.ds(c * _CHUNK, _CHUNK)]],
                rows_v.at[b], gsem[b])

        def compute(c, b):
            def group_body(g, carry):
                rbase = g * _G
                masks = []
                for r in range(_G):
                    lane_idx = jnp.full((_LANES,), c * _CHUNK + rbase + r,
                                        jnp.int32)
                    t = plsc.load_gather(ttid_v, [lane_idx])
                    masks.append(t != 0)

                zeros = jnp.zeros((_LANES,), jnp.float32)

                @plsc.parallel_loop(
                    0, _NSLICE, unroll=2,
                    carry=(tuple([zeros] * _G), tuple([zeros] * _G)))
                def s1_loop(s, acc):
                    accs, accqs = acc
                    sl = pl.ds(s * _LANES, _LANES)
                    tt0 = tt_v[0, sl]
                    tt1 = tt_v[1, sl]
                    new_a = []
                    new_q = []
                    for r in range(_G):
                        x = rows_v[b, rbase + r, sl] + jnp.where(
                            masks[r], tt1, tt0)
                        rows_v[b, rbase + r, sl] = x
                        new_a.append(accs[r] + x)
                        new_q.append(accqs[r] + x * x)
                    return tuple(new_a), tuple(new_q)

                accs, accqs = s1_loop

                means = []
                rvs = []
                for r in range(_G):
                    mean = jnp.sum(accs[r]) * (1.0 / _HIDDEN)
                    var = jnp.maximum(
                        jnp.sum(accqs[r]) * (1.0 / _HIDDEN) - mean * mean,
                        0.0)
                    means.append(jnp.full((_LANES,), mean))
                    rvs.append(_rsqrt_vec(jnp.full((_LANES,), var + _EPS)))

                @plsc.parallel_loop(0, _NSLICE, unroll=2, carry=jnp.int32(0))
                def s2_loop(s, carry2):
                    sl = pl.ds(s * _LANES, _LANES)
                    gm = g_v[sl]
                    bt = b_v[sl]
                    for r in range(_G):
                        x = rows_v[b, rbase + r, sl]
                        rows_v[b, rbase + r, sl] = (
                            (x - means[r]) * rvs[r] * gm + bt)
                    return carry2

                del s2_loop
                return carry

            lax.fori_loop(0, _NG, group_body, jnp.int32(0))

        gh = [None] * _NCHUNK
        sh = [None] * _NCHUNK
        gh[0] = start_gather(0, 0)
        for c in range(_NCHUNK):
            b = c & 1
            nb = b ^ 1
            if c + 1 < _NCHUNK:
                if c - 1 >= 0:
                    sh[c - 1].wait()
                gh[c + 1] = start_gather(c + 1, nb)
            gh[c].wait()
            compute(c, b)
            sh[c] = pltpu.async_copy(
                rows_v.at[b],
                out_hbm.at[pl.ds(tbase + c * _CHUNK, _CHUNK)], ssem[b])
        sh[_NCHUNK - 2].wait()
        sh[_NCHUNK - 1].wait()

    return emb_kernel


_EMB_KERNEL = _make_kernel()


def kernel(input_ids, token_type_ids, word_emb, token_type_emb, ln_gamma,
           ln_beta):
    b, l = input_ids.shape
    ids = input_ids.reshape(b * l).astype(jnp.int32)
    ttids = token_type_ids.reshape(b * l).astype(jnp.int32)
    out = _EMB_KERNEL(ids, ttids, word_emb, token_type_emb, ln_gamma,
                      ln_beta)
    return out.reshape(b, l, word_emb.shape[1])
